# baseline (speedup 1.0000x reference)
import jax
import jax.numpy as jnp
from jax.experimental import pallas as pl


def _elu_kernel(x_ref, o_ref):
    v = x_ref[...]
    o_ref[...] = jnp.where(v > 0, v, jnp.exp(jnp.minimum(v, 0.0)) - 1.0)


def _gatv2(h, src, dst, ea, Wl, Wr, We, att, b, N):
    xl = h @ Wl
    xr = h @ Wr
    m = xl[src] + xr[dst] + ea @ We
    e = jnp.where(m > 0, m, 0.2 * m)
    alpha = jnp.sum(e * att, axis=-1)
    amax = jax.ops.segment_max(alpha, dst, num_segments=N)
    amax = jnp.where(jnp.isfinite(amax), amax, 0.0)
    ex = jnp.exp(alpha - amax[dst])
    den = jax.ops.segment_sum(ex, dst, num_segments=N)
    a = ex / (den[dst] + 1e-16)
    out = jax.ops.segment_sum(a[:, None] * xl[src], dst, num_segments=N)
    return out + b


def _elu(x):
    return pl.pallas_call(
        _elu_kernel,
        out_shape=jax.ShapeDtypeStruct(x.shape, x.dtype),
    )(x)


def kernel(x, edge_index, pos, Wl0, Wr0, We0, att0, b0, Wl1, Wr1, We1, att1, b1, Wl2, Wr2, We2, att2, b2):
    N = x.shape[0]
    src = edge_index[0]
    dst = edge_index[1]
    ea = pos[dst] - pos[src]
    h = _elu(_gatv2(jnp.concatenate((x, pos), axis=1), src, dst, ea, Wl0, Wr0, We0, att0, b0, N))
    h = _elu(_gatv2(jnp.concatenate((h, pos), axis=1), src, dst, ea, Wl1, Wr1, We1, att1, b1, N))
    out = _elu(_gatv2(jnp.concatenate((h, pos), axis=1), src, dst, ea, Wl2, Wr2, We2, att2, b2, N))
    return out


# speedup vs baseline: 2.3141x; 2.3141x over previous
import functools

import jax
import jax.numpy as jnp
from jax import lax
from jax.experimental import pallas as pl
from jax.experimental.pallas import tpu as pltpu
from jax.experimental.pallas import tpu_sc as plsc

N = 10000
E = 320000
NC = 2
NS = 16
NW = NC * NS
C = 80
NCHUNK = E // C
ITERS = NCHUNK // NW
NP = 10240
RS = NP // NS
DR = NP // 128
TW = 128
BR = 1024


def _sc_edge_kernel(do):
    mesh = plsc.VectorSubcoreMesh(core_axis_name="c", subcore_axis_name="s")

    @functools.partial(
        pl.kernel,
        out_type=(
            jax.ShapeDtypeStruct((NC, NP, TW), jnp.float32),
            jax.ShapeDtypeStruct((NC, DR, 128), jnp.float32),
        ),
        mesh=mesh,
        compiler_params=pltpu.CompilerParams(needs_layout_passes=False),
        scratch_types=[
            pltpu.VMEM((C,), jnp.int32),
            pltpu.VMEM((C,), jnp.int32),
            pltpu.VMEM((C, TW), jnp.float32),
            pltpu.VMEM((C, TW), jnp.float32),
            pltpu.VMEM((C, TW), jnp.float32),
            pltpu.VMEM((do,), jnp.float32),
            pltpu.VMEM((DR, 128), jnp.float32),
            pltpu.VMEM((DR,), jnp.int32),
            pltpu.VMEM_SHARED((NP, TW), jnp.float32),
            pltpu.VMEM_SHARED((DR, 128), jnp.float32),
            pltpu.SemaphoreType.DMA,
        ],
    )
    def edge_kernel(a_hbm, b_hbm, xl_hbm, src_hbm, dst_hbm, att_hbm, z_hbm,
                    num_out, den_out, idx_src, idx_dst, rows_a, rows_b,
                    rows_xl, att_v, den_local, row_ids, num_sh, den_sh, sem):
        c = lax.axis_index("c")
        s = lax.axis_index("s")
        wid = s * NC + c

        pltpu.sync_copy(z_hbm.at[pl.ds(s * RS, RS)],
                        num_sh.at[pl.ds(s * RS, RS)])

        @pl.when(s < 10)
        def _():
            pltpu.sync_copy(z_hbm.at[pl.ds(s * 8, 8)],
                            den_sh.at[pl.ds(s * 8, 8)])

        pltpu.sync_copy(att_hbm, att_v)

        def zden_body(r, carry):
            for k in range(8):
                den_local[r, pl.ds(k * 16, 16)] = jnp.zeros((16,), jnp.float32)
            return carry

        lax.fori_loop(0, DR, zden_body, 0)

        def rid_body(j, carry):
            row_ids[pl.ds(j * 16, 16)] = lax.iota(jnp.int32, 16) + j * 16
            return carry

        lax.fori_loop(0, DR // 16, rid_body, 0)
        plsc.subcore_barrier()

        def chunk_body(i, carry):
            cid = wid + i * NW
            if True:
                base = cid * C
                pltpu.sync_copy(src_hbm.at[pl.ds(base, C)], idx_src)
                pltpu.sync_copy(dst_hbm.at[pl.ds(base, C)], idx_dst)
                ca = pltpu.async_copy(a_hbm.at[idx_src], rows_a, sem)
                cb = pltpu.async_copy(b_hbm.at[idx_dst], rows_b, sem)
                cx = pltpu.async_copy(xl_hbm.at[idx_src], rows_xl, sem)
                ca.wait()
                cb.wait()
                cx.wait()

                def group_body(g, carry2):
                    ev = lax.iota(jnp.int32, 16) + g * 16

                    def alpha_body(k, acc):
                        kv = jnp.broadcast_to(k, (16,))
                        m = (plsc.load_gather(rows_a, [ev, kv])
                             + plsc.load_gather(rows_b, [ev, kv]))
                        lr = jnp.maximum(m, 0.2 * m)
                        av = plsc.load_gather(att_v, [kv])
                        return acc + lr * av

                    acc = lax.fori_loop(0, do, alpha_body,
                                        jnp.zeros((16,), jnp.float32))
                    exv = jnp.exp(acc)

                    def scale_body(k, carry3):
                        kv = jnp.broadcast_to(k, (16,))
                        v = plsc.load_gather(rows_xl, [ev, kv])
                        plsc.store_scatter(rows_xl, [ev, kv], v * exv)
                        return carry3

                    lax.fori_loop(0, do, scale_body, 0)
                    dstv = idx_dst[pl.ds(g * 16, 16)]
                    plsc.addupdate_scatter(
                        den_local, [dstv >> 7, dstv & 127], exv)
                    return carry2

                lax.fori_loop(0, C // 16, group_body, 0)
                pltpu.sync_copy(rows_xl, num_sh.at[idx_dst], add=True)

            return carry

        lax.fori_loop(0, ITERS, chunk_body, 0)
        pltpu.sync_copy(den_local, den_sh.at[row_ids], add=True)
        plsc.subcore_barrier()

        pltpu.sync_copy(num_sh.at[pl.ds(s * RS, RS)],
                        num_out.at[c, pl.ds(s * RS, RS)])

        @pl.when(s < 10)
        def _():
            pltpu.sync_copy(den_sh.at[pl.ds(s * 8, 8)],
                            den_out.at[c, pl.ds(s * 8, 8)])

    return edge_kernel


_sc_edge_128 = _sc_edge_kernel(128)
_sc_edge_64 = _sc_edge_kernel(64)


def _tc_layer(first):
    def body(*refs):
        if first:
            (h_ref, pos_ref, wlh_ref, wap_ref, wrh_ref, wbp_ref,
             xle_ref, a_ref, b_ref) = refs
            h = h_ref[...]
        else:
            (np_ref, dp_ref, bias_ref, pos_ref, wlh_ref, wap_ref, wrh_ref,
             wbp_ref, xle_ref, a_ref, b_ref) = refs
            num = np_ref[0] + np_ref[1]
            den = dp_ref[0] + dp_ref[1]
            h = num / (den[:, None] + 1e-16) + bias_ref[...]
            h = jnp.where(h > 0, h, jnp.exp(jnp.minimum(h, 0.0)) - 1.0)
        p = pos_ref[...]
        xl = (jnp.dot(h, wlh_ref[...], preferred_element_type=jnp.float32)
              + jnp.dot(p, wap_ref[0], preferred_element_type=jnp.float32))
        pa = jnp.dot(p, wap_ref[1], preferred_element_type=jnp.float32)
        xr = (jnp.dot(h, wrh_ref[...], preferred_element_type=jnp.float32)
              + jnp.dot(p, wbp_ref[0], preferred_element_type=jnp.float32))
        xle_ref[...] = xl
        a_ref[...] = xl - pa
        b_ref[...] = xr

    return body


def _run_tc_layer(h_or_numP, denF, bias, pos, Wlh, Wap2, Wrh, Wbp2, first):
    grid = (NP // BR,)
    full = lambda shape: pl.BlockSpec(shape, lambda i: (0,) * len(shape))
    w_specs = [full((TW, TW)), full((2, 3, TW)), full((TW, TW)),
               full((1, 3, TW))]
    row = lambda w: pl.BlockSpec((BR, w), lambda i: (i, 0))
    if first:
        in_specs = [row(TW), row(3)] + w_specs
        args = (h_or_numP, pos, Wlh, Wap2, Wrh, Wbp2)
    else:
        in_specs = [
            pl.BlockSpec((NC, BR, TW), lambda i: (0, i, 0)),
            pl.BlockSpec((NC, BR), lambda i: (0, i)),
            full((TW,)),
            row(3),
        ] + w_specs
        args = (h_or_numP, denF, bias, pos, Wlh, Wap2, Wrh, Wbp2)
    out_shapes = [jax.ShapeDtypeStruct((NP, TW), jnp.float32)] * 3
    out_specs = [row(TW)] * 3
    return pl.pallas_call(
        _tc_layer(first),
        grid=grid,
        in_specs=in_specs,
        out_specs=out_specs,
        out_shape=out_shapes,
    )(*args)


def _tc_final_body(np_ref, dp_ref, bias_ref, out_ref):
    num = np_ref[0] + np_ref[1]
    den = dp_ref[0] + dp_ref[1]
    h = num[:, :64] / (den[:, None] + 1e-16) + bias_ref[...]
    out_ref[...] = jnp.where(h > 0, h, jnp.exp(jnp.minimum(h, 0.0)) - 1.0)


def _run_tc_final(numP, denF, bias):
    return pl.pallas_call(
        _tc_final_body,
        grid=(NP // BR,),
        in_specs=[
            pl.BlockSpec((NC, BR, TW), lambda i: (0, i, 0)),
            pl.BlockSpec((NC, BR), lambda i: (0, i)),
            pl.BlockSpec((64,), lambda i: (0,)),
        ],
        out_specs=pl.BlockSpec((BR, 64), lambda i: (i, 0)),
        out_shape=jax.ShapeDtypeStruct((NP, 64), jnp.float32),
    )(numP, denF, bias)


def _prep(Wl, Wr, We, do):
    Wlh, Wlp = Wl[:TW], Wl[TW:]
    Wrh, Wrp = Wr[:TW], Wr[TW:]
    if do < TW:
        pad = [(0, 0), (0, TW - do)]
        Wlh, Wlp = jnp.pad(Wlh, pad), jnp.pad(Wlp, pad)
        Wrh, Wrp = jnp.pad(Wrh, pad), jnp.pad(Wrp, pad)
        We = jnp.pad(We, pad)
    Wap2 = jnp.stack([Wlp, We])
    Wbp2 = (Wrp + We)[None]
    return Wlh, Wap2, Wrh, Wbp2


def kernel(x, edge_index, pos, Wl0, Wr0, We0, att0, b0, Wl1, Wr1, We1, att1,
           b1, Wl2, Wr2, We2, att2, b2):
    src = edge_index[0]
    dst = edge_index[1]
    xp = jnp.pad(x, [(0, NP - N), (0, 0)])
    posp = jnp.pad(pos, [(0, NP - N), (0, 0)])
    z = jnp.zeros((NP, TW), jnp.float32)

    Wlh, Wap2, Wrh, Wbp2 = _prep(Wl0, Wr0, We0, 128)
    xle, a, b = _run_tc_layer(xp, None, None, posp, Wlh, Wap2, Wrh, Wbp2, True)
    numP, denP = _sc_edge_128(a, b, xle, src, dst, att0, z)

    Wlh, Wap2, Wrh, Wbp2 = _prep(Wl1, Wr1, We1, 128)
    xle, a, b = _run_tc_layer(numP, denP.reshape(NC, NP), b0, posp, Wlh, Wap2,
                              Wrh, Wbp2, False)
    numP, denP = _sc_edge_128(a, b, xle, src, dst, att1, z)

    Wlh, Wap2, Wrh, Wbp2 = _prep(Wl2, Wr2, We2, 64)
    xle, a, b = _run_tc_layer(numP, denP.reshape(NC, NP), b1, posp, Wlh, Wap2,
                              Wrh, Wbp2, False)
    numP, denP = _sc_edge_64(a, b, xle, src, dst, att2, z)

    out = _run_tc_final(numP, denP.reshape(NC, NP), b2)
    return out[:N]


# speedup vs baseline: 2.3510x; 1.0159x over previous
import functools

import jax
import jax.numpy as jnp
from jax import lax
from jax.experimental import pallas as pl
from jax.experimental.pallas import tpu as pltpu
from jax.experimental.pallas import tpu_sc as plsc

N = 10000
E = 320000
NC = 2
NS = 16
NW = NC * NS
C = 80
NCHUNK = E // C
ITERS = NCHUNK // NW
NP = 10240
RS = NP // NS
DR = NP // 128
TW = 128
BR = 1024


def _sc_edge_kernel(do):
    mesh = plsc.VectorSubcoreMesh(core_axis_name="c", subcore_axis_name="s")

    @functools.partial(
        pl.kernel,
        out_type=(
            jax.ShapeDtypeStruct((NC, NP, TW), jnp.float32),
            jax.ShapeDtypeStruct((NC, DR, 128), jnp.float32),
        ),
        mesh=mesh,
        compiler_params=pltpu.CompilerParams(needs_layout_passes=False),
        scratch_types=[
            pltpu.VMEM((C,), jnp.int32),
            pltpu.VMEM((C,), jnp.int32),
            pltpu.VMEM((C, TW), jnp.float32),
            pltpu.VMEM((C, TW), jnp.float32),
            pltpu.VMEM((C, TW), jnp.float32),
            pltpu.VMEM((do,), jnp.float32),
            pltpu.VMEM((DR, 128), jnp.float32),
            pltpu.VMEM((DR,), jnp.int32),
            pltpu.VMEM_SHARED((NP, TW), jnp.float32),
            pltpu.VMEM_SHARED((DR, 128), jnp.float32),
            pltpu.SemaphoreType.DMA,
        ],
    )
    def edge_kernel(a_hbm, b_hbm, xl_hbm, src_hbm, dst_hbm, att_hbm, z_hbm,
                    num_out, den_out, idx_src, idx_dst, rows_a, rows_b,
                    rows_xl, att_v, den_local, row_ids, num_sh, den_sh, sem):
        c = lax.axis_index("c")
        s = lax.axis_index("s")
        wid = s * NC + c

        pltpu.sync_copy(z_hbm.at[pl.ds(s * RS, RS)],
                        num_sh.at[pl.ds(s * RS, RS)])

        @pl.when(s < 10)
        def _():
            pltpu.sync_copy(z_hbm.at[pl.ds(s * 8, 8)],
                            den_sh.at[pl.ds(s * 8, 8)])

        pltpu.sync_copy(att_hbm, att_v)

        def zden_body(r, carry):
            for k in range(8):
                den_local[r, pl.ds(k * 16, 16)] = jnp.zeros((16,), jnp.float32)
            return carry

        lax.fori_loop(0, DR, zden_body, 0)

        def rid_body(j, carry):
            row_ids[pl.ds(j * 16, 16)] = lax.iota(jnp.int32, 16) + j * 16
            return carry

        lax.fori_loop(0, DR // 16, rid_body, 0)
        plsc.subcore_barrier()

        def chunk_body(i, carry):
            cid = wid + i * NW
            if True:
                base = cid * C
                pltpu.sync_copy(src_hbm.at[pl.ds(base, C)], idx_src)
                pltpu.sync_copy(dst_hbm.at[pl.ds(base, C)], idx_dst)
                ca = pltpu.async_copy(a_hbm.at[idx_src], rows_a, sem)
                cb = pltpu.async_copy(b_hbm.at[idx_dst], rows_b, sem)
                cx = pltpu.async_copy(xl_hbm.at[idx_src], rows_xl, sem)
                ca.wait()
                cb.wait()
                cx.wait()

                def group_body(g, carry2):
                    ev = lax.iota(jnp.int32, 16) + g * 16

                    def alpha_blk(k16, acc):
                        base = k16 * 16
                        for dk in range(16):
                            kv = jnp.broadcast_to(base + dk, (16,))
                            m = (plsc.load_gather(rows_a, [ev, kv])
                                 + plsc.load_gather(rows_b, [ev, kv]))
                            lr = jnp.maximum(m, 0.2 * m)
                            acc = acc + lr * plsc.load_gather(att_v, [kv])
                        return acc

                    acc = lax.fori_loop(0, do // 16, alpha_blk,
                                        jnp.zeros((16,), jnp.float32))
                    exv = jnp.exp(acc)

                    def scale_blk(k16, carry3):
                        base = k16 * 16
                        for dk in range(16):
                            kv = jnp.broadcast_to(base + dk, (16,))
                            v = plsc.load_gather(rows_xl, [ev, kv])
                            plsc.store_scatter(rows_xl, [ev, kv], v * exv)
                        return carry3

                    lax.fori_loop(0, do // 16, scale_blk, 0)
                    dstv = idx_dst[pl.ds(g * 16, 16)]
                    plsc.addupdate_scatter(
                        den_local, [dstv >> 7, dstv & 127], exv)
                    return carry2

                lax.fori_loop(0, C // 16, group_body, 0)
                pltpu.sync_copy(rows_xl, num_sh.at[idx_dst], add=True)

            return carry

        lax.fori_loop(0, ITERS, chunk_body, 0)
        pltpu.sync_copy(den_local, den_sh.at[row_ids], add=True)
        plsc.subcore_barrier()

        pltpu.sync_copy(num_sh.at[pl.ds(s * RS, RS)],
                        num_out.at[c, pl.ds(s * RS, RS)])

        @pl.when(s < 10)
        def _():
            pltpu.sync_copy(den_sh.at[pl.ds(s * 8, 8)],
                            den_out.at[c, pl.ds(s * 8, 8)])

    return edge_kernel


_sc_edge_128 = _sc_edge_kernel(128)
_sc_edge_64 = _sc_edge_kernel(64)


def _tc_layer(first):
    def body(*refs):
        if first:
            (h_ref, pos_ref, wlh_ref, wap_ref, wrh_ref, wbp_ref,
             xle_ref, a_ref, b_ref) = refs
            h = h_ref[...]
        else:
            (np_ref, dp_ref, bias_ref, pos_ref, wlh_ref, wap_ref, wrh_ref,
             wbp_ref, xle_ref, a_ref, b_ref) = refs
            num = np_ref[0] + np_ref[1]
            den = dp_ref[0] + dp_ref[1]
            h = num / (den[:, None] + 1e-16) + bias_ref[...]
            h = jnp.where(h > 0, h, jnp.exp(jnp.minimum(h, 0.0)) - 1.0)
        p = pos_ref[...]
        xl = (jnp.dot(h, wlh_ref[...], preferred_element_type=jnp.float32)
              + jnp.dot(p, wap_ref[0], preferred_element_type=jnp.float32))
        pa = jnp.dot(p, wap_ref[1], preferred_element_type=jnp.float32)
        xr = (jnp.dot(h, wrh_ref[...], preferred_element_type=jnp.float32)
              + jnp.dot(p, wbp_ref[0], preferred_element_type=jnp.float32))
        xle_ref[...] = xl
        a_ref[...] = xl - pa
        b_ref[...] = xr

    return body


def _run_tc_layer(h_or_numP, denF, bias, pos, Wlh, Wap2, Wrh, Wbp2, first):
    grid = (NP // BR,)
    full = lambda shape: pl.BlockSpec(shape, lambda i: (0,) * len(shape))
    w_specs = [full((TW, TW)), full((2, 3, TW)), full((TW, TW)),
               full((1, 3, TW))]
    row = lambda w: pl.BlockSpec((BR, w), lambda i: (i, 0))
    if first:
        in_specs = [row(TW), row(3)] + w_specs
        args = (h_or_numP, pos, Wlh, Wap2, Wrh, Wbp2)
    else:
        in_specs = [
            pl.BlockSpec((NC, BR, TW), lambda i: (0, i, 0)),
            pl.BlockSpec((NC, BR), lambda i: (0, i)),
            full((TW,)),
            row(3),
        ] + w_specs
        args = (h_or_numP, denF, bias, pos, Wlh, Wap2, Wrh, Wbp2)
    out_shapes = [jax.ShapeDtypeStruct((NP, TW), jnp.float32)] * 3
    out_specs = [row(TW)] * 3
    return pl.pallas_call(
        _tc_layer(first),
        grid=grid,
        in_specs=in_specs,
        out_specs=out_specs,
        out_shape=out_shapes,
    )(*args)


def _tc_final_body(np_ref, dp_ref, bias_ref, out_ref):
    num = np_ref[0] + np_ref[1]
    den = dp_ref[0] + dp_ref[1]
    h = num[:, :64] / (den[:, None] + 1e-16) + bias_ref[...]
    out_ref[...] = jnp.where(h > 0, h, jnp.exp(jnp.minimum(h, 0.0)) - 1.0)


def _run_tc_final(numP, denF, bias):
    return pl.pallas_call(
        _tc_final_body,
        grid=(NP // BR,),
        in_specs=[
            pl.BlockSpec((NC, BR, TW), lambda i: (0, i, 0)),
            pl.BlockSpec((NC, BR), lambda i: (0, i)),
            pl.BlockSpec((64,), lambda i: (0,)),
        ],
        out_specs=pl.BlockSpec((BR, 64), lambda i: (i, 0)),
        out_shape=jax.ShapeDtypeStruct((NP, 64), jnp.float32),
    )(numP, denF, bias)


def _prep(Wl, Wr, We, do):
    Wlh, Wlp = Wl[:TW], Wl[TW:]
    Wrh, Wrp = Wr[:TW], Wr[TW:]
    if do < TW:
        pad = [(0, 0), (0, TW - do)]
        Wlh, Wlp = jnp.pad(Wlh, pad), jnp.pad(Wlp, pad)
        Wrh, Wrp = jnp.pad(Wrh, pad), jnp.pad(Wrp, pad)
        We = jnp.pad(We, pad)
    Wap2 = jnp.stack([Wlp, We])
    Wbp2 = (Wrp + We)[None]
    return Wlh, Wap2, Wrh, Wbp2


def kernel(x, edge_index, pos, Wl0, Wr0, We0, att0, b0, Wl1, Wr1, We1, att1,
           b1, Wl2, Wr2, We2, att2, b2):
    src = edge_index[0]
    dst = edge_index[1]
    xp = jnp.pad(x, [(0, NP - N), (0, 0)])
    posp = jnp.pad(pos, [(0, NP - N), (0, 0)])
    z = jnp.zeros((NP, TW), jnp.float32)

    Wlh, Wap2, Wrh, Wbp2 = _prep(Wl0, Wr0, We0, 128)
    xle, a, b = _run_tc_layer(xp, None, None, posp, Wlh, Wap2, Wrh, Wbp2, True)
    numP, denP = _sc_edge_128(a, b, xle, src, dst, att0, z)

    Wlh, Wap2, Wrh, Wbp2 = _prep(Wl1, Wr1, We1, 128)
    xle, a, b = _run_tc_layer(numP, denP.reshape(NC, NP), b0, posp, Wlh, Wap2,
                              Wrh, Wbp2, False)
    numP, denP = _sc_edge_128(a, b, xle, src, dst, att1, z)

    Wlh, Wap2, Wrh, Wbp2 = _prep(Wl2, Wr2, We2, 64)
    xle, a, b = _run_tc_layer(numP, denP.reshape(NC, NP), b1, posp, Wlh, Wap2,
                              Wrh, Wbp2, False)
    numP, denP = _sc_edge_64(a, b, xle, src, dst, att2, z)

    out = _run_tc_final(numP, denP.reshape(NC, NP), b2)
    return out[:N]
